# Initial kernel scaffold; baseline (speedup 1.0000x reference)
import jax
import jax.numpy as jnp
from jax.experimental import pallas as pl


def kernel(likelihood_position, likelihood_count, local_cellxregion_ix):
    raise NotImplementedError("write your pallas kernel here")


# speedup vs baseline: 2.6020x; 2.6020x over previous
import functools

import jax
import jax.numpy as jnp
from jax import lax
from jax.experimental import pallas as pl
from jax.experimental.pallas import tpu as pltpu
from jax.experimental.pallas import tpu_sc as plsc

_N_CELLS = 200
_N_REGIONS = 500
_NSEG = _N_CELLS * _N_REGIONS
_BC = 6272
_NSEG_PAD = 16 * _BC
_F = 6400000
_NW = 32
_PER_TILE = _F // _NW
_CF = 4000
_NCHUNK = _PER_TILE // _CF
_VPC = _CF // 16

_mesh = plsc.VectorSubcoreMesh(core_axis_name="c", subcore_axis_name="s")


@functools.partial(
    pl.kernel,
    mesh=_mesh,
    out_type=jax.ShapeDtypeStruct((_NW, _NSEG_PAD), jnp.float32),
    scratch_types=[
        pltpu.VMEM((_NSEG_PAD,), jnp.float32),
        pltpu.VMEM((_CF,), jnp.float32),
        pltpu.VMEM((_CF,), jnp.int32),
    ],
    compiler_params=pltpu.CompilerParams(needs_layout_passes=False),
)
def _sc_partial_sums(vals_hbm, idx_hbm, part_hbm, acc_v, vals_v, idx_v):
    c = lax.axis_index("c")
    s = lax.axis_index("s")
    wid = s * 2 + c
    base = wid * _PER_TILE

    zeros = jnp.zeros((16,), jnp.float32)

    def _zero(i, carry):
        acc_v[pl.ds(i * 16, 16)] = zeros
        return carry

    lax.fori_loop(0, _NSEG_PAD // 16, _zero, 0, unroll=8)

    def _chunk(g, carry):
        off = base + g * _CF
        pltpu.sync_copy(vals_hbm.at[pl.ds(off, _CF)], vals_v)
        pltpu.sync_copy(idx_hbm.at[pl.ds(off, _CF)], idx_v)

        def _vreg(i, c2):
            ix = idx_v[pl.ds(i * 16, 16)]
            x = vals_v[pl.ds(i * 16, 16)]
            plsc.addupdate_scatter(acc_v, [ix], x)
            return c2

        lax.fori_loop(0, _VPC, _vreg, 0, unroll=4)
        return carry

    lax.fori_loop(0, _NCHUNK, _chunk, 0)

    pltpu.sync_copy(acc_v, part_hbm.at[wid])


def _merge_body(part_ref, cnt_ref, out_ref):
    out_ref[0, 0, :] = jnp.sum(part_ref[...], axis=0) + cnt_ref[0, 0, :]


def _tc_merge(part, cnt3):
    return pl.pallas_call(
        _merge_body,
        grid=(_NSEG_PAD // _BC,),
        in_specs=[
            pl.BlockSpec((_NW, _BC), lambda i: (0, i)),
            pl.BlockSpec((1, 1, _BC), lambda i: (i, 0, 0)),
        ],
        out_specs=pl.BlockSpec((1, 1, _BC), lambda i: (i, 0, 0)),
        out_shape=jax.ShapeDtypeStruct((_NSEG_PAD // _BC, 1, _BC), jnp.float32),
    )(part, cnt3)


def kernel(likelihood_position, likelihood_count, local_cellxregion_ix):
    part = _sc_partial_sums(likelihood_position, local_cellxregion_ix)
    cnt3 = jnp.pad(likelihood_count, (0, _NSEG_PAD - _NSEG)).reshape(
        _NSEG_PAD // _BC, 1, _BC
    )
    out3 = _tc_merge(part, cnt3)
    return out3.reshape(-1)[:_NSEG].reshape(_N_CELLS, _N_REGIONS)


# speedup vs baseline: 5.2186x; 2.0056x over previous
import functools

import jax
import jax.numpy as jnp
from jax import lax
from jax.experimental import pallas as pl
from jax.experimental.pallas import tpu as pltpu
from jax.experimental.pallas import tpu_sc as plsc

_N_CELLS = 200
_N_REGIONS = 500
_NSEG = _N_CELLS * _N_REGIONS
_BC = 6272
_NSEG_PAD = 16 * _BC
_F = 6400000
_NW = 32
_PER_TILE = _F // _NW
_CF = 4000
_NCHUNK = _PER_TILE // _CF
_VPC = _CF // 16

_mesh = plsc.VectorSubcoreMesh(core_axis_name="c", subcore_axis_name="s")


@functools.partial(
    pl.kernel,
    mesh=_mesh,
    out_type=jax.ShapeDtypeStruct((_NW, _NSEG_PAD), jnp.float32),
    scratch_types=[
        pltpu.VMEM((_NSEG_PAD,), jnp.float32),
        pltpu.VMEM((_CF,), jnp.float32),
        pltpu.VMEM((_CF,), jnp.float32),
        pltpu.VMEM((_CF + 16,), jnp.int32),
        pltpu.VMEM((_CF + 16,), jnp.int32),
        pltpu.SemaphoreType.DMA,
        pltpu.SemaphoreType.DMA,
        pltpu.SemaphoreType.DMA,
        pltpu.SemaphoreType.DMA,
    ],
    compiler_params=pltpu.CompilerParams(needs_layout_passes=False),
)
def _sc_partial_sums(vals_hbm, idx_hbm, part_hbm, acc_v, vals0, vals1,
                     idx0, idx1, sv0, sv1, si0, si1):
    c = lax.axis_index("c")
    s = lax.axis_index("s")
    wid = s * 2 + c
    base = wid * _PER_TILE
    svs = (sv0, sv1)
    sis = (si0, si1)
    vbufs = (vals0, vals1)
    ibufs = (idx0, idx1)

    zeros = jnp.zeros((16,), jnp.float32)

    def _zero(i, carry):
        acc_v[pl.ds(i * 16, 16)] = zeros
        return carry

    lax.fori_loop(0, _NSEG_PAD // 16, _zero, 0, unroll=8)

    lane = lax.iota(jnp.int32, 16)
    is_last_lane = lane == 15

    def _copies(g, b):
        off = base + g * _CF
        cv = pltpu.make_async_copy(
            vals_hbm.at[pl.ds(off, _CF)], vbufs[b], svs[b])
        ci = pltpu.make_async_copy(
            idx_hbm.at[pl.ds(off, _CF)], ibufs[b].at[pl.ds(0, _CF)], sis[b])
        return cv, ci

    def _start(g, b):
        cv, ci = _copies(g, b)
        cv.start()
        ci.start()

    def _wait(g, b):
        cv, ci = _copies(g, b)
        cv.wait()
        ci.wait()

    def _compute(g, b):
        _wait(g, b)

        def _vreg(i, c2):
            ix = ibufs[b][pl.ds(i * 16, 16)]
            nx = ibufs[b][pl.ds(i * 16 + 1, 16)]
            x = vbufs[b][pl.ds(i * 16, 16)]
            csum = plsc.cumsum(x)
            boundary = ix != nx
            m_end = boundary | is_last_lane
            m_carry = boundary & (~is_last_lane)
            plsc.addupdate_scatter(acc_v, [ix], csum, mask=m_end)
            plsc.addupdate_scatter(acc_v, [nx], -csum, mask=m_carry)
            return c2

        lax.fori_loop(0, _VPC, _vreg, 0, unroll=4)

    izeros = jnp.zeros((16,), jnp.int32)
    idx0[pl.ds(_CF, 16)] = izeros
    idx1[pl.ds(_CF, 16)] = izeros

    _start(0, 0)

    def _outer(gg, carry):
        g0 = gg * 2

        @pl.when(g0 + 1 < _NCHUNK)
        def _():
            _start(g0 + 1, 1)

        _compute(g0, 0)

        @pl.when(g0 + 2 < _NCHUNK)
        def _():
            _start(g0 + 2, 0)

        @pl.when(g0 + 1 < _NCHUNK)
        def _():
            _compute(g0 + 1, 1)

        return carry

    lax.fori_loop(0, (_NCHUNK + 1) // 2, _outer, 0)

    pltpu.sync_copy(acc_v, part_hbm.at[wid])


def _merge_body(part_ref, cnt_ref, out_ref):
    out_ref[0, 0, :] = jnp.sum(part_ref[...], axis=0) + cnt_ref[0, 0, :]


def _tc_merge(part, cnt3):
    return pl.pallas_call(
        _merge_body,
        grid=(_NSEG_PAD // _BC,),
        in_specs=[
            pl.BlockSpec((_NW, _BC), lambda i: (0, i)),
            pl.BlockSpec((1, 1, _BC), lambda i: (i, 0, 0)),
        ],
        out_specs=pl.BlockSpec((1, 1, _BC), lambda i: (i, 0, 0)),
        out_shape=jax.ShapeDtypeStruct((_NSEG_PAD // _BC, 1, _BC), jnp.float32),
    )(part, cnt3)


def kernel(likelihood_position, likelihood_count, local_cellxregion_ix):
    part = _sc_partial_sums(likelihood_position, local_cellxregion_ix)
    cnt3 = jnp.pad(likelihood_count, (0, _NSEG_PAD - _NSEG)).reshape(
        _NSEG_PAD // _BC, 1, _BC
    )
    out3 = _tc_merge(part, cnt3)
    return out3.reshape(-1)[:_NSEG].reshape(_N_CELLS, _N_REGIONS)


# speedup vs baseline: 11.3153x; 2.1683x over previous
import functools

import jax
import jax.numpy as jnp
from jax import lax
from jax.experimental import pallas as pl
from jax.experimental.pallas import tpu as pltpu
from jax.experimental.pallas import tpu_sc as plsc

_N_CELLS = 200
_N_REGIONS = 500
_NSEG = _N_CELLS * _N_REGIONS
_BC = 6272
_NSEG_PAD = 16 * _BC
_F = 6400000
_NW = 32
_PER_TILE = _F // _NW
_CF = 4000
_NCHUNK = _PER_TILE // _CF
_VPC = _CF // 16

_mesh = plsc.VectorSubcoreMesh(core_axis_name="c", subcore_axis_name="s")


@functools.partial(
    pl.kernel,
    mesh=_mesh,
    out_type=jax.ShapeDtypeStruct((_NW, _NSEG_PAD), jnp.float32),
    scratch_types=[
        pltpu.VMEM((_NSEG_PAD,), jnp.float32),
        pltpu.VMEM((_CF,), jnp.float32),
        pltpu.VMEM((_CF,), jnp.float32),
        pltpu.VMEM((_CF + 16,), jnp.int32),
        pltpu.VMEM((_CF + 16,), jnp.int32),
        pltpu.SemaphoreType.DMA,
        pltpu.SemaphoreType.DMA,
        pltpu.SemaphoreType.DMA,
        pltpu.SemaphoreType.DMA,
    ],
    compiler_params=pltpu.CompilerParams(needs_layout_passes=False),
)
def _sc_partial_sums(vals_hbm, idx_hbm, part_hbm, acc_v, vals0, vals1,
                     idx0, idx1, sv0, sv1, si0, si1):
    c = lax.axis_index("c")
    s = lax.axis_index("s")
    wid = s * 2 + c
    base = wid * _PER_TILE
    svs = (sv0, sv1)
    sis = (si0, si1)
    vbufs = (vals0, vals1)
    ibufs = (idx0, idx1)

    zeros = jnp.zeros((16,), jnp.float32)

    def _zero(i, carry):
        acc_v[pl.ds(i * 16, 16)] = zeros
        return carry

    lax.fori_loop(0, _NSEG_PAD // 16, _zero, 0, unroll=8)

    lane = lax.iota(jnp.int32, 16)
    is_last_lane = lane == 15

    def _copies(g, b):
        off = base + g * _CF
        cv = pltpu.make_async_copy(
            vals_hbm.at[pl.ds(off, _CF)], vbufs[b], svs[b])
        ci = pltpu.make_async_copy(
            idx_hbm.at[pl.ds(off, _CF)], ibufs[b].at[pl.ds(0, _CF)], sis[b])
        return cv, ci

    def _start(g, b):
        cv, ci = _copies(g, b)
        cv.start()
        ci.start()

    def _wait(g, b):
        cv, ci = _copies(g, b)
        cv.wait()
        ci.wait()

    def _compute(g, b):
        _wait(g, b)

        @plsc.parallel_loop(0, _VPC, unroll=8)
        def _vreg(i):
            ix = ibufs[b][pl.ds(i * 16, 16)]
            nx = ibufs[b][pl.ds(i * 16 + 1, 16)]
            x = vbufs[b][pl.ds(i * 16, 16)]
            csum = plsc.cumsum(x)
            boundary = ix != nx
            m_end = boundary | is_last_lane
            m_carry = boundary & (~is_last_lane)
            plsc.addupdate_scatter(acc_v, [ix], csum, mask=m_end)
            plsc.addupdate_scatter(acc_v, [nx], -csum, mask=m_carry)

    izeros = jnp.zeros((16,), jnp.int32)
    idx0[pl.ds(_CF, 16)] = izeros
    idx1[pl.ds(_CF, 16)] = izeros

    _start(0, 0)

    def _outer(gg, carry):
        g0 = gg * 2

        @pl.when(g0 + 1 < _NCHUNK)
        def _():
            _start(g0 + 1, 1)

        _compute(g0, 0)

        @pl.when(g0 + 2 < _NCHUNK)
        def _():
            _start(g0 + 2, 0)

        @pl.when(g0 + 1 < _NCHUNK)
        def _():
            _compute(g0 + 1, 1)

        return carry

    lax.fori_loop(0, (_NCHUNK + 1) // 2, _outer, 0)

    pltpu.sync_copy(acc_v, part_hbm.at[wid])


def _merge_body(part_ref, cnt_ref, out_ref):
    out_ref[0, 0, :] = jnp.sum(part_ref[...], axis=0) + cnt_ref[0, 0, :]


def _tc_merge(part, cnt3):
    return pl.pallas_call(
        _merge_body,
        grid=(_NSEG_PAD // _BC,),
        in_specs=[
            pl.BlockSpec((_NW, _BC), lambda i: (0, i)),
            pl.BlockSpec((1, 1, _BC), lambda i: (i, 0, 0)),
        ],
        out_specs=pl.BlockSpec((1, 1, _BC), lambda i: (i, 0, 0)),
        out_shape=jax.ShapeDtypeStruct((_NSEG_PAD // _BC, 1, _BC), jnp.float32),
    )(part, cnt3)


def kernel(likelihood_position, likelihood_count, local_cellxregion_ix):
    part = _sc_partial_sums(likelihood_position, local_cellxregion_ix)
    cnt3 = jnp.pad(likelihood_count, (0, _NSEG_PAD - _NSEG)).reshape(
        _NSEG_PAD // _BC, 1, _BC
    )
    out3 = _tc_merge(part, cnt3)
    return out3.reshape(-1)[:_NSEG].reshape(_N_CELLS, _N_REGIONS)
